# baseline (speedup 1.0000x reference)
import functools

import jax
import jax.numpy as jnp
from jax import lax
from jax.experimental import pallas as pl
from jax.experimental.pallas import tpu as pltpu
from jax.experimental.pallas import tpu_sc as plsc

_NUM_CORES = 2
_NUM_SUBCORES = 16
_NUM_WORKERS = _NUM_CORES * _NUM_SUBCORES

_CHUNK = 512
_LANES = 128
_SUB = 8


def _make_lookup(batch: int, seq: int, vocab: int, dim: int):
    total = batch * seq
    per_worker = total // _NUM_WORKERS
    num_chunks = per_worker // _CHUNK
    chunks_per_s = batch // _CHUNK
    tj_n = dim // _SUB
    tv_per_chunk = _CHUNK // _LANES
    slab = tv_per_chunk * _SUB * _LANES
    minor = (batch // _LANES) * _SUB * _LANES
    assert num_chunks % 2 == 0 and dim == 32 and _CHUNK % _LANES == 0

    mesh = plsc.VectorSubcoreMesh(core_axis_name="c", subcore_axis_name="s")

    @functools.partial(
        pl.kernel,
        mesh=mesh,
        out_type=jax.ShapeDtypeStruct((seq, tj_n, minor), jnp.float32),
        scratch_types=[
            pltpu.VMEM((per_worker,), jnp.int32),
            [pltpu.VMEM((_CHUNK, dim), jnp.float32) for _ in range(4)],
            [pltpu.VMEM((tj_n * slab,), jnp.float32) for _ in range(2)],
            [pltpu.SemaphoreType.DMA for _ in range(4)],
            [pltpu.SemaphoreType.DMA for _ in range(2)],
        ],
        compiler_params=pltpu.CompilerParams(
            use_tc_tiling_on_sc=False, needs_layout_passes=False),
    )
    def lookup(idx_hbm, table_hbm, out_hbm, idx_v, rows, trans, gsems, osems):
        wid = lax.axis_index("s") * _NUM_CORES + lax.axis_index("c")
        qbase = wid * num_chunks
        pltpu.sync_copy(idx_hbm.at[pl.ds(wid * per_worker, per_worker)], idx_v)

        lane = lax.iota(jnp.int32, 16)
        dst_pat = []
        for c in range(dim // 16):
            j = lane + (c * 16)
            dst_pat.append((j >> 3) * slab + (j & 7) * _LANES)

        def start_gather(i, p):
            pltpu.async_copy(
                table_hbm.at[idx_v.at[pl.ds(i * _CHUNK, _CHUNK)]],
                rows[p], gsems[p])

        def wait_gather(i, p):
            pltpu.make_async_copy(
                table_hbm.at[idx_v.at[pl.ds(i * _CHUNK, _CHUNK)]],
                rows[p], gsems[p]).wait()

        def out_copies(i, p):
            q = qbase + i
            s = q // chunks_per_s
            tv0 = (q % chunks_per_s) * tv_per_chunk
            return [
                pltpu.make_async_copy(
                    trans[p].at[pl.ds(tj * slab, slab)],
                    out_hbm.at[s, tj, pl.ds(tv0 * _LANES * _SUB, slab)],
                    osems[p])
                for tj in range(tj_n)
            ]

        def transpose_chunk(g, p):
            rows_p, trans_p = rows[g], trans[p]
            n_c = dim // 16

            def body(m, carry):
                b0 = (m >> 3) * (_SUB * _LANES) + (m & 7) * 16
                dst_m = [dst_pat[c] + b0 for c in range(n_c)]
                vals = []
                for k in range(16):
                    t = m * 16 + k
                    for c in range(n_c):
                        vals.append(rows_p[t, pl.ds(c * 16, 16)])
                for k in range(16):
                    for c in range(n_c):
                        plsc.store_scatter(
                            trans_p, [dst_m[c] + k], vals[k * n_c + c])
                return carry

            lax.fori_loop(0, _CHUNK // 16, body, 0)

        def step(i, g, p):
            wait_gather(i, g)

            @pl.when(i >= 2)
            def _():
                for cp in out_copies(i - 2, p):
                    cp.wait()

            transpose_chunk(g, p)
            for cp in out_copies(i, p):
                cp.start()

            @pl.when(i + 4 < num_chunks)
            def _():
                start_gather(i + 4, g)

        for g in range(4):
            start_gather(g, g)

        def outer(t, carry):
            for g in range(4):
                step(4 * t + g, g, g % 2)
            return carry

        n_full = (num_chunks // 4) * 4
        lax.fori_loop(0, num_chunks // 4, outer, 0)
        for i in range(n_full, num_chunks):
            step(i, i % 4, i % 2)

        for cp in out_copies(num_chunks - 2, 0):
            cp.wait()
        for cp in out_copies(num_chunks - 1, 1):
            cp.wait()

    return lookup


def kernel(token_ids, weights):
    batch, seq = token_ids.shape
    vocab, dim = weights.shape
    flat_idx = token_ids.T.reshape(batch * seq).astype(jnp.int32)
    lookup = _make_lookup(batch, seq, vocab, dim)
    out = lookup(flat_idx, weights)
    out5d = out.reshape(seq, dim // _SUB, batch // _LANES, _SUB, _LANES)
    return out5d.transpose(2, 4, 0, 1, 3).reshape(batch, seq, dim)


# speedup vs baseline: 1.5020x; 1.5020x over previous
import functools

import jax
import jax.numpy as jnp
from jax import lax
from jax.experimental import pallas as pl
from jax.experimental.pallas import tpu as pltpu
from jax.experimental.pallas import tpu_sc as plsc

_NUM_CORES = 2
_NUM_SUBCORES = 16
_NUM_WORKERS = _NUM_CORES * _NUM_SUBCORES

_CHUNK = 512
_LANES = 128
_SUB = 8


def _make_lookup(batch: int, seq: int, vocab: int, dim: int):
    total = batch * seq
    per_worker = total // _NUM_WORKERS
    num_chunks = per_worker // _CHUNK
    chunks_per_s = batch // _CHUNK
    tj_n = dim // _SUB
    tv_per_chunk = _CHUNK // _LANES
    slab = tv_per_chunk * _SUB * _LANES
    minor = (batch // _LANES) * _SUB * _LANES
    assert num_chunks % 2 == 0 and dim == 32 and _CHUNK % _LANES == 0

    mesh = plsc.VectorSubcoreMesh(core_axis_name="c", subcore_axis_name="s")

    @functools.partial(
        pl.kernel,
        mesh=mesh,
        out_type=jax.ShapeDtypeStruct((seq, tj_n, minor), jnp.float32),
        scratch_types=[
            pltpu.VMEM((per_worker,), jnp.int32),
            [pltpu.VMEM((_CHUNK, dim), jnp.float32) for _ in range(4)],
            pltpu.VMEM((_LANES * (dim + 1),), jnp.float32),
            [pltpu.VMEM((tj_n * slab,), jnp.float32) for _ in range(2)],
            [pltpu.SemaphoreType.DMA for _ in range(4)],
            [pltpu.SemaphoreType.DMA for _ in range(2)],
        ],
        compiler_params=pltpu.CompilerParams(
            use_tc_tiling_on_sc=False, needs_layout_passes=False),
    )
    def lookup(idx_hbm, table_hbm, out_hbm, idx_v, rows, rpad, trans,
               gsems, osems):
        wid = lax.axis_index("s") * _NUM_CORES + lax.axis_index("c")
        qbase = wid * num_chunks
        pltpu.sync_copy(idx_hbm.at[pl.ds(wid * per_worker, per_worker)], idx_v)

        lane = lax.iota(jnp.int32, 16)

        def start_gather(i, p):
            pltpu.async_copy(
                table_hbm.at[idx_v.at[pl.ds(i * _CHUNK, _CHUNK)]],
                rows[p], gsems[p])

        def wait_gather(i, p):
            pltpu.make_async_copy(
                table_hbm.at[idx_v.at[pl.ds(i * _CHUNK, _CHUNK)]],
                rows[p], gsems[p]).wait()

        def out_copies(i, p):
            q = qbase + i
            s = q // chunks_per_s
            tv0 = (q % chunks_per_s) * tv_per_chunk
            return [
                pltpu.make_async_copy(
                    trans[p].at[pl.ds(tj * slab, slab)],
                    out_hbm.at[s, tj, pl.ds(tv0 * _LANES * _SUB, slab)],
                    osems[p])
                for tj in range(tj_n)
            ]

        def transpose_chunk(g, p):
            rows_p, trans_p = rows[g], trans[p]
            n_c = dim // 16
            stride = dim + 1
            iota33 = lane * stride

            def tv_body(tv, carry):
                t0 = tv * _LANES

                def repack(m2, carry2):
                    vals = []
                    for k in range(16):
                        t = t0 + m2 * 16 + k
                        for c in range(n_c):
                            vals.append(rows_p[t, pl.ds(c * 16, 16)])
                    for k in range(16):
                        tl = m2 * 16 + k
                        for c in range(n_c):
                            rpad[pl.ds(tl * stride + c * 16, 16)] = (
                                vals[k * n_c + c])
                    return carry2

                lax.fori_loop(0, _LANES // 16, repack, 0)

                def trans_body(c2, carry2):
                    base = c2 * (16 * stride)
                    vals = []
                    for j in range(dim):
                        vals.append(
                            plsc.load_gather(rpad, [iota33 + (base + j)]))
                    dbase = tv * (_SUB * _LANES) + c2 * 16
                    for j in range(dim):
                        tj, u = j >> 3, j & 7
                        trans_p[pl.ds(tj * slab + u * _LANES + dbase, 16)] = (
                            vals[j])
                    return carry2

                lax.fori_loop(0, _LANES // 16, trans_body, 0)
                return carry

            lax.fori_loop(0, tv_per_chunk, tv_body, 0)

        def step(i, g, p):
            wait_gather(i, g)

            @pl.when(i >= 2)
            def _():
                for cp in out_copies(i - 2, p):
                    cp.wait()

            transpose_chunk(g, p)
            for cp in out_copies(i, p):
                cp.start()

            @pl.when(i + 4 < num_chunks)
            def _():
                start_gather(i + 4, g)

        for g in range(4):
            start_gather(g, g)

        def outer(t, carry):
            for g in range(4):
                step(4 * t + g, g, g % 2)
            return carry

        n_full = (num_chunks // 4) * 4
        lax.fori_loop(0, num_chunks // 4, outer, 0)
        for i in range(n_full, num_chunks):
            step(i, i % 4, i % 2)

        for cp in out_copies(num_chunks - 2, 0):
            cp.wait()
        for cp in out_copies(num_chunks - 1, 1):
            cp.wait()

    return lookup


def kernel(token_ids, weights):
    batch, seq = token_ids.shape
    vocab, dim = weights.shape
    flat_idx = token_ids.T.reshape(batch * seq).astype(jnp.int32)
    lookup = _make_lookup(batch, seq, vocab, dim)
    out = lookup(flat_idx, weights)
    out5d = out.reshape(seq, dim // _SUB, batch // _LANES, _SUB, _LANES)
    return out5d.transpose(2, 4, 0, 1, 3).reshape(batch, seq, dim)


# speedup vs baseline: 3.1291x; 2.0833x over previous
import functools

import jax
import jax.numpy as jnp
from jax import lax
from jax.experimental import pallas as pl
from jax.experimental.pallas import tpu as pltpu
from jax.experimental.pallas import tpu_sc as plsc

_NUM_CORES = 2
_NUM_SUBCORES = 16
_NUM_WORKERS = _NUM_CORES * _NUM_SUBCORES

_CHUNK = 512
_LANES = 128
_SUB = 8


def _make_lookup(batch: int, seq: int, vocab: int, dim: int):
    total = batch * seq
    per_worker = total // _NUM_WORKERS
    num_chunks = per_worker // _CHUNK
    chunks_per_s = batch // _CHUNK
    tj_n = dim // _SUB
    tv_per_chunk = _CHUNK // _LANES
    slab = tv_per_chunk * _SUB * _LANES
    minor = (batch // _LANES) * _SUB * _LANES
    assert num_chunks % 2 == 0 and dim == 32 and _CHUNK % _LANES == 0

    mesh = plsc.VectorSubcoreMesh(core_axis_name="c", subcore_axis_name="s")

    @functools.partial(
        pl.kernel,
        mesh=mesh,
        out_type=jax.ShapeDtypeStruct((seq, tj_n, minor), jnp.float32),
        scratch_types=[
            pltpu.VMEM((per_worker,), jnp.int32),
            [pltpu.VMEM((_CHUNK, dim), jnp.float32) for _ in range(4)],
            pltpu.VMEM((_LANES * (dim + 1),), jnp.float32),
            [pltpu.VMEM((tj_n * slab,), jnp.float32) for _ in range(2)],
            [pltpu.SemaphoreType.DMA for _ in range(4)],
            [pltpu.SemaphoreType.DMA for _ in range(2)],
        ],
        compiler_params=pltpu.CompilerParams(
            use_tc_tiling_on_sc=False, needs_layout_passes=False),
    )
    def lookup(idx_hbm, table_hbm, out_hbm, idx_v, rows, rpad, trans,
               gsems, osems):
        wid = lax.axis_index("s") * _NUM_CORES + lax.axis_index("c")
        qbase = wid * num_chunks
        pltpu.sync_copy(idx_hbm.at[pl.ds(wid * per_worker, per_worker)], idx_v)

        lane = lax.iota(jnp.int32, 16)

        def start_gather(i, p):
            pltpu.async_copy(
                table_hbm.at[idx_v.at[pl.ds(i * _CHUNK, _CHUNK)]],
                rows[p], gsems[p])

        def wait_gather(i, p):
            pltpu.make_async_copy(
                table_hbm.at[idx_v.at[pl.ds(i * _CHUNK, _CHUNK)]],
                rows[p], gsems[p]).wait()

        def out_copies(i, p):
            q = qbase + i
            s = q // chunks_per_s
            tv0 = (q % chunks_per_s) * tv_per_chunk
            return [
                pltpu.make_async_copy(
                    trans[p].at[pl.ds(tj * slab, slab)],
                    out_hbm.at[s, tj, pl.ds(tv0 * _LANES * _SUB, slab)],
                    osems[p])
                for tj in range(tj_n)
            ]

        def transpose_chunk(g, p):
            rows_p, trans_p = rows[g], trans[p]
            n_c = dim // 16
            stride = dim + 1
            iota33 = lane * stride

            def tv_body(tv, carry):
                t0 = tv * _LANES

                def repack(m2, carry2):
                    vals = []
                    for k in range(16):
                        t = t0 + m2 * 16 + k
                        for c in range(n_c):
                            vals.append(rows_p[t, pl.ds(c * 16, 16)])
                    for k in range(16):
                        tl = m2 * 16 + k
                        for c in range(n_c):
                            rpad[pl.ds(tl * stride + c * 16, 16)] = (
                                vals[k * n_c + c])
                    return carry2

                lax.fori_loop(0, _LANES // 16, repack, 0)

                def trans_body(c2, carry2):
                    base = c2 * (16 * stride)
                    vals = []
                    for j in range(dim):
                        vals.append(
                            plsc.load_gather(rpad, [iota33 + (base + j)]))
                    dbase = tv * (_SUB * _LANES) + c2 * 16
                    for j in range(dim):
                        tj, u = j >> 3, j & 7
                        trans_p[pl.ds(tj * slab + u * _LANES + dbase, 16)] = (
                            vals[j])
                    return carry2

                lax.fori_loop(0, _LANES // 16, trans_body, 0)
                return carry

            lax.fori_loop(0, tv_per_chunk, tv_body, 0)

        def step(i, g, p):
            wait_gather(i, g)

            @pl.when(i >= 2)
            def _():
                for cp in out_copies(i - 2, p):
                    cp.wait()

            transpose_chunk(g, p)
            for cp in out_copies(i, p):
                cp.start()

            @pl.when(i + 4 < num_chunks)
            def _():
                start_gather(i + 4, g)

        for g in range(4):
            start_gather(g, g)

        def outer(t, carry):
            for g in range(4):
                step(4 * t + g, g, g % 2)
            return carry

        n_full = (num_chunks // 4) * 4
        lax.fori_loop(0, num_chunks // 4, outer, 0)
        for i in range(n_full, num_chunks):
            step(i, i % 4, i % 2)

        for cp in out_copies(num_chunks - 2, 0):
            cp.wait()
        for cp in out_copies(num_chunks - 1, 1):
            cp.wait()

    return lookup


_VBLK = 512


def _make_detile(vocab: int, dim: int):
    n_full = (vocab // _VBLK) - (1 if vocab % _VBLK else 0)
    assert n_full % _NUM_WORKERS == 0
    per_w = n_full // _NUM_WORKERS
    rem_v0 = n_full * _VBLK
    rem_full = (vocab - rem_v0) // _LANES
    rem_tail = vocab - rem_v0 - rem_full * _LANES

    mesh = plsc.VectorSubcoreMesh(core_axis_name="c", subcore_axis_name="s")

    @functools.partial(
        pl.kernel,
        mesh=mesh,
        out_type=jax.ShapeDtypeStruct((vocab * dim,), jnp.float32),
        scratch_types=[
            [pltpu.VMEM((dim, _VBLK), jnp.float32) for _ in range(2)],
            pltpu.VMEM((_LANES * (dim + 1),), jnp.float32),
            [pltpu.VMEM((_VBLK * dim,), jnp.float32) for _ in range(2)],
            [pltpu.SemaphoreType.DMA for _ in range(2)],
            [pltpu.SemaphoreType.DMA for _ in range(2)],
        ],
        compiler_params=pltpu.CompilerParams(needs_layout_passes=False),
    )
    def detile(tableT_hbm, tail_hbm, lin_hbm, inblk, rpad, linout,
               isems, osems):
        wid = lax.axis_index("s") * _NUM_CORES + lax.axis_index("c")
        lane = lax.iota(jnp.int32, 16)
        stride = dim + 1
        iota33 = lane * stride
        n_c = dim // 16

        def blk(t):
            return (wid + t * _NUM_WORKERS) * _VBLK

        def start_in(v0, nv, p):
            pltpu.async_copy(
                tableT_hbm.at[:, pl.ds(v0, nv)],
                inblk[p].at[:, pl.ds(0, nv)], isems[p])

        def wait_in(v0, nv, p):
            pltpu.make_async_copy(
                tableT_hbm.at[:, pl.ds(v0, nv)],
                inblk[p].at[:, pl.ds(0, nv)], isems[p]).wait()

        def start_out(v0, nv, p):
            pltpu.async_copy(
                linout[p].at[pl.ds(0, nv * dim)],
                lin_hbm.at[pl.ds(v0 * dim, nv * dim)], osems[p])

        def wait_out(v0, nv, p):
            pltpu.make_async_copy(
                linout[p].at[pl.ds(0, nv * dim)],
                lin_hbm.at[pl.ds(v0 * dim, nv * dim)], osems[p]).wait()

        def transpose_block(p, n_tv):
            in_p, out_p = inblk[p], linout[p]

            def tv_body(tv, carry):
                v0 = tv * _LANES

                def repack(c2, carry2):
                    base = c2 * 16
                    vals = []
                    for j in range(dim):
                        vals.append(in_p[j, pl.ds(v0 + base, 16)])
                    sbase = base * stride
                    for j in range(dim):
                        plsc.store_scatter(
                            rpad, [iota33 + (sbase + j)], vals[j])
                    return carry2

                lax.fori_loop(0, _LANES // 16, repack, 0)

                def unspread(m2, carry2):
                    vals = []
                    for k in range(16):
                        vl = m2 * 16 + k
                        for c in range(n_c):
                            vals.append(rpad[pl.ds(vl * stride + c * 16, 16)])
                    obase = (v0 + m2 * 16) * dim
                    for k in range(16):
                        for c in range(n_c):
                            out_p[pl.ds(obase + k * dim + c * 16, 16)] = (
                                vals[k * n_c + c])
                    return carry2

                lax.fori_loop(0, _LANES // 16, unspread, 0)
                return carry

            lax.fori_loop(0, n_tv, tv_body, 0)

        def step(t, p):
            wait_in(blk(t), _VBLK, p)

            @pl.when(t + 1 < per_w)
            def _():
                start_in(blk(t + 1), _VBLK, 1 - p)

            @pl.when(t >= 2)
            def _():
                wait_out(blk(t - 2), _VBLK, p)

            transpose_block(p, _VBLK // _LANES)
            start_out(blk(t), _VBLK, p)

        start_in(blk(0), _VBLK, 0)

        def outer(q, carry):
            step(2 * q, 0)
            step(2 * q + 1, 1)
            return carry

        lax.fori_loop(0, per_w // 2, outer, 0)
        for t in range((per_w // 2) * 2, per_w):
            step(t, t % 2)
        wait_out(blk(per_w - 2), _VBLK, (per_w - 2) % 2)
        wait_out(blk(per_w - 1), _VBLK, (per_w - 1) % 2)

        @pl.when(wid == 0)
        def _():
            if rem_full:
                nv = rem_full * _LANES
                start_in(rem_v0, nv, 0)
                wait_in(rem_v0, nv, 0)
                transpose_block(0, rem_full)
                start_out(rem_v0, nv, 0)
                wait_out(rem_v0, nv, 0)
            if rem_tail:
                n = rem_tail * dim
                v0 = rem_v0 + rem_full * _LANES
                pltpu.sync_copy(tail_hbm, linout[1].at[pl.ds(0, n)])
                pltpu.sync_copy(linout[1].at[pl.ds(0, n)],
                                lin_hbm.at[pl.ds(v0 * dim, n)])

    return detile


def kernel(token_ids, weights):
    batch, seq = token_ids.shape
    vocab, dim = weights.shape
    flat_idx = token_ids.T.reshape(batch * seq).astype(jnp.int32)
    n_main = (vocab // _VBLK - (1 if vocab % _VBLK else 0)) * _VBLK
    rem_full = (vocab - n_main) // 128
    tail_v0 = n_main + rem_full * 128
    tail_lin = weights[tail_v0:, :].reshape(-1)
    lin_table = _make_detile(vocab, dim)(weights.T, tail_lin)
    lookup = _make_lookup(batch, seq, vocab, dim)
    out = lookup(flat_idx, lin_table.reshape(vocab, dim))
    out5d = out.reshape(seq, dim // _SUB, batch // _LANES, _SUB, _LANES)
    return out5d.transpose(2, 4, 0, 1, 3).reshape(batch, seq, dim)
